# baseline (speedup 1.0000x reference)
import functools

import jax
import jax.numpy as jnp
from jax import lax
from jax.experimental import pallas as pl
from jax.experimental.pallas import tpu as pltpu
from jax.experimental.pallas import tpu_sc as plsc

_N = 10000
_D_IN = 128
_D_H = 256
_N_CLS = 64
_E = 320000

_NC = 2
_NS = 16
_CH = 128
_SUB = 3
_CPT = 156
_BLOCKS = _CPT // _SUB
_HALFB = _BLOCKS // 2
_XTRA = _E // _CH - _NS * _CPT
_N_PAD = 10240
_OPT = 624


def _agg_body(table, edges, zrows, z16, ones16, out_sum, out_cnt,
              src_v, dst_v, rows_v, ones_v, acc, cnt, sem0, sem1, *, d2):
    c = lax.axis_index("c")
    s = lax.axis_index("s")
    wid = c * _NS + s
    sems = (sem0, sem1)

    rpt = _N_PAD // _NS
    r0 = s * rpt
    pltpu.sync_copy(zrows.at[pl.ds(r0, rpt)], acc.at[pl.ds(r0, rpt)])
    pltpu.sync_copy(z16.at[pl.ds(r0, rpt)], cnt.at[pl.ds(r0, rpt)])
    pltpu.sync_copy(ones16, ones_v)
    plsc.subcore_barrier()

    chunk0 = s * _CPT
    tbl = table.at[c]

    def _fetch(k, blk, nch, chunk_base):
        pltpu.sync_copy(edges.at[0, pl.ds(chunk_base, nch)],
                        src_v.at[k, pl.ds(0, nch)])
        pltpu.sync_copy(edges.at[1, pl.ds(chunk_base, nch)],
                        dst_v.at[k, pl.ds(0, nch)])
        for j in range(nch):
            pltpu.async_copy(tbl.at[src_v.at[k, j]], rows_v.at[k, j],
                             sems[k])

    def _drain_scatter(k, nch, docnt):
        for j in range(nch):
            pltpu.make_async_copy(tbl.at[src_v.at[k, j]],
                                  rows_v.at[k, j], sems[k]).wait()
        for j in range(nch):
            pltpu.sync_copy(rows_v.at[k, j], acc.at[dst_v.at[k, j]],
                            add=True)

        @pl.when(docnt)
        def _():
            for j in range(nch):
                pltpu.sync_copy(ones_v, cnt.at[dst_v.at[k, j]], add=True)

    for k in range(2):
        _fetch(k, k, _SUB, chunk0 + k * _SUB)

    def body(io, carry):
        for k in range(2):
            i = io * 2 + k
            docnt = jnp.where(c == 0, i < _HALFB, i >= _HALFB)
            _drain_scatter(k, _SUB, docnt)

            @pl.when(i + 2 < _BLOCKS)
            def _():
                _fetch(k, i + 2, _SUB, chunk0 + (i + 2) * _SUB)
        return carry

    lax.fori_loop(0, _BLOCKS // 2, body, 0)

    @pl.when(s < _XTRA)
    def _():
        _fetch(0, 0, 1, _NS * _CPT + s)
        _drain_scatter(0, 1, c == 1)

    plsc.subcore_barrier()

    o0 = s * _OPT
    rem0 = (_NS - 1) * _OPT
    rem = _N - rem0

    @pl.when(s < _NS - 1)
    def _():
        pltpu.sync_copy(acc.at[pl.ds(o0, _OPT)], out_sum.at[c, pl.ds(o0, _OPT)])
        pltpu.sync_copy(cnt.at[pl.ds(o0, _OPT)], out_cnt.at[c, pl.ds(o0, _OPT)])

    @pl.when(s == _NS - 1)
    def _():
        pltpu.sync_copy(acc.at[pl.ds(rem0, rem)], out_sum.at[c, pl.ds(rem0, rem)])
        pltpu.sync_copy(cnt.at[pl.ds(rem0, rem)], out_cnt.at[c, pl.ds(rem0, rem)])


def _make_agg(d2):
    mesh = plsc.VectorSubcoreMesh(core_axis_name="c", subcore_axis_name="s",
                                  num_cores=_NC, num_subcores=_NS)
    return functools.partial(
        pl.kernel,
        mesh=mesh,
        out_type=[jax.ShapeDtypeStruct((_NC, _N, d2), jnp.float32),
                  jax.ShapeDtypeStruct((_NC, _N, 16), jnp.float32)],
        scratch_types=[
            pltpu.VMEM((2, _SUB, _CH), jnp.int32),
            pltpu.VMEM((2, _SUB, _CH), jnp.int32),
            pltpu.VMEM((2, _SUB, _CH, d2), jnp.float32),
            pltpu.VMEM((_CH, 16), jnp.float32),
            pltpu.VMEM_SHARED((_N_PAD, d2), jnp.float32),
            pltpu.VMEM_SHARED((_N_PAD, 16), jnp.float32),
            pltpu.SemaphoreType.DMA,
            pltpu.SemaphoreType.DMA,
        ],
        compiler_params=pltpu.CompilerParams(use_tc_tiling_on_sc=False),
        name=f"sage_segsum_d{d2}",
    )(functools.partial(_agg_body, d2=d2))


_agg64 = _make_agg(_D_IN // _NC)
_agg32 = _make_agg(_N_CLS // _NC)


def _layer_tc_body(p_ref, c_ref, w0_ref, b0_ref, w1_ref, b1_ref, z_ref):
    ssum = jnp.concatenate([p_ref[0], p_ref[1]], axis=-1)
    cntv = c_ref[0, :, 0:1] + c_ref[1, :, 0:1]
    mean = ssum / jnp.maximum(cntv, 1.0)
    mask = (cntv > 0.0).astype(jnp.float32)
    h = jnp.dot(mean, w0_ref[...], preferred_element_type=jnp.float32)
    h = jnp.maximum(h + b0_ref[...] * mask, 0.0)
    z_ref[...] = (jnp.dot(h, w1_ref[...], preferred_element_type=jnp.float32)
                  + b1_ref[...])


def _mean_tc_body(p_ref, c_ref, o_ref):
    ssum = jnp.concatenate([p_ref[0], p_ref[1]], axis=-1)
    cntv = c_ref[0, :, 0:1] + c_ref[1, :, 0:1]
    o_ref[...] = ssum / jnp.maximum(cntv, 1.0)


_BM = 1000


def _layer_tc(p, c, w0, b0, w1, b1):
    return pl.pallas_call(
        _layer_tc_body,
        grid=(_N // _BM,),
        in_specs=[
            pl.BlockSpec((_NC, _BM, _D_IN // _NC), lambda i: (0, i, 0)),
            pl.BlockSpec((_NC, _BM, 16), lambda i: (0, i, 0)),
            pl.BlockSpec((_D_IN, _D_H), lambda i: (0, 0)),
            pl.BlockSpec((1, _D_H), lambda i: (0, 0)),
            pl.BlockSpec((_D_H, _N_CLS), lambda i: (0, 0)),
            pl.BlockSpec((1, _N_CLS), lambda i: (0, 0)),
        ],
        out_specs=pl.BlockSpec((_BM, _N_CLS), lambda i: (i, 0)),
        out_shape=jax.ShapeDtypeStruct((_N, _N_CLS), jnp.float32),
        name="sage_dense",
    )(p, c, w0, b0, w1, b1)


def _mean_tc(p, c):
    return pl.pallas_call(
        _mean_tc_body,
        grid=(_N // _BM,),
        in_specs=[
            pl.BlockSpec((_NC, _BM, _N_CLS // _NC), lambda i: (0, i, 0)),
            pl.BlockSpec((_NC, _BM, 16), lambda i: (0, i, 0)),
        ],
        out_specs=pl.BlockSpec((_BM, _N_CLS), lambda i: (i, 0)),
        out_shape=jax.ShapeDtypeStruct((_N, _N_CLS), jnp.float32),
        name="sage_mean",
    )(p, c)


def kernel(x, edge_index0, edge_index1, W0, b0, W1, b1):
    z64 = jnp.zeros((_N_PAD, _D_IN // _NC), jnp.float32)
    z32 = jnp.zeros((_N_PAD, _N_CLS // _NC), jnp.float32)
    z16 = jnp.zeros((_N_PAD, 16), jnp.float32)
    ones16 = jnp.ones((_CH, 16), jnp.float32)

    d2 = _D_IN // _NC
    x3 = jnp.stack([x[:, :d2], x[:, d2:]])
    e0r = edge_index0.reshape(2, _E // _CH, _CH)
    e1r = edge_index1.reshape(2, _E // _CH, _CH)
    p0, c0 = _agg64(x3, e0r, z64, z16, ones16)
    z = _layer_tc(p0, c0, W0, b0.reshape(1, -1), W1, b1.reshape(1, -1))
    d2z = _N_CLS // _NC
    z3 = jnp.stack([z[:, :d2z], z[:, d2z:]])
    p1, c1 = _agg32(z3, e1r, z32, z16, ones16)
    return _mean_tc(p1, c1)


# speedup vs baseline: 1.0774x; 1.0774x over previous
import functools

import jax
import jax.numpy as jnp
from jax import lax
from jax.experimental import pallas as pl
from jax.experimental.pallas import tpu as pltpu
from jax.experimental.pallas import tpu_sc as plsc

_N = 10000
_D_IN = 128
_D_H = 256
_N_CLS = 64
_E = 320000

_NC = 2
_NS = 16
_CH = 128
_SUB = 4
_CPT = 156
_BLOCKS = _CPT // _SUB
_HALFB = _BLOCKS // 2
_XTRA = _E // _CH - _NS * _CPT
_N_PAD = 10240
_OPT = 624


def _agg_body(table, edges, zrows, z16, ones16, out_sum, out_cnt,
              src_v, dst_v, rows_v, ones_v, acc, cnt, sem0, sem1, *, d2):
    c = lax.axis_index("c")
    s = lax.axis_index("s")
    wid = c * _NS + s
    sems = (sem0, sem1)

    rpt = _N_PAD // _NS
    r0 = s * rpt
    pltpu.sync_copy(zrows.at[pl.ds(r0, rpt)], acc.at[pl.ds(r0, rpt)])
    pltpu.sync_copy(z16.at[pl.ds(r0, rpt)], cnt.at[pl.ds(r0, rpt)])
    pltpu.sync_copy(ones16, ones_v)
    plsc.subcore_barrier()

    chunk0 = s * _CPT
    tbl = table.at[c]

    def _fetch(k, blk, nch, chunk_base):
        pltpu.sync_copy(edges.at[0, pl.ds(chunk_base, nch)],
                        src_v.at[k, pl.ds(0, nch)])
        pltpu.sync_copy(edges.at[1, pl.ds(chunk_base, nch)],
                        dst_v.at[k, pl.ds(0, nch)])
        for j in range(nch):
            pltpu.async_copy(tbl.at[src_v.at[k, j]], rows_v.at[k, j],
                             sems[k])

    def _drain_scatter(k, nch, docnt):
        for j in range(nch):
            pltpu.make_async_copy(tbl.at[src_v.at[k, j]],
                                  rows_v.at[k, j], sems[k]).wait()
        for j in range(nch):
            pltpu.sync_copy(rows_v.at[k, j], acc.at[dst_v.at[k, j]],
                            add=True)

        @pl.when(docnt)
        def _():
            for j in range(nch):
                pltpu.sync_copy(ones_v, cnt.at[dst_v.at[k, j]], add=True)

    for k in range(2):
        _fetch(k, k, _SUB, chunk0 + k * _SUB)

    def body(io, carry):
        for k in range(2):
            i = io * 2 + k
            docnt = jnp.where(c == 0, i < _HALFB, i >= _HALFB)
            _drain_scatter(k, _SUB, docnt)

            @pl.when(i + 2 < _BLOCKS)
            def _():
                _fetch(k, i + 2, _SUB, chunk0 + (i + 2) * _SUB)
        return carry

    lax.fori_loop(0, (_BLOCKS - 1) // 2, body, 0)
    _drain_scatter(0, _SUB, c == 1)

    @pl.when(s == 0)
    def _():
        _fetch(0, 0, _XTRA, _NS * _CPT)
        _drain_scatter(0, _XTRA, c == 1)

    plsc.subcore_barrier()

    o0 = s * _OPT
    rem0 = (_NS - 1) * _OPT
    rem = _N - rem0

    @pl.when(s < _NS - 1)
    def _():
        pltpu.sync_copy(acc.at[pl.ds(o0, _OPT)], out_sum.at[c, pl.ds(o0, _OPT)])
        pltpu.sync_copy(cnt.at[pl.ds(o0, _OPT)], out_cnt.at[c, pl.ds(o0, _OPT)])

    @pl.when(s == _NS - 1)
    def _():
        pltpu.sync_copy(acc.at[pl.ds(rem0, rem)], out_sum.at[c, pl.ds(rem0, rem)])
        pltpu.sync_copy(cnt.at[pl.ds(rem0, rem)], out_cnt.at[c, pl.ds(rem0, rem)])


def _make_agg(d2):
    mesh = plsc.VectorSubcoreMesh(core_axis_name="c", subcore_axis_name="s",
                                  num_cores=_NC, num_subcores=_NS)
    return functools.partial(
        pl.kernel,
        mesh=mesh,
        out_type=[jax.ShapeDtypeStruct((_NC, _N, d2), jnp.float32),
                  jax.ShapeDtypeStruct((_NC, _N, 16), jnp.float32)],
        scratch_types=[
            pltpu.VMEM((2, _SUB, _CH), jnp.int32),
            pltpu.VMEM((2, _SUB, _CH), jnp.int32),
            pltpu.VMEM((2, _SUB, _CH, d2), jnp.float32),
            pltpu.VMEM((_CH, 16), jnp.float32),
            pltpu.VMEM_SHARED((_N_PAD, d2), jnp.float32),
            pltpu.VMEM_SHARED((_N_PAD, 16), jnp.float32),
            pltpu.SemaphoreType.DMA,
            pltpu.SemaphoreType.DMA,
        ],
        compiler_params=pltpu.CompilerParams(use_tc_tiling_on_sc=False),
        name=f"sage_segsum_d{d2}",
    )(functools.partial(_agg_body, d2=d2))


_agg64 = _make_agg(_D_IN // _NC)
_agg32 = _make_agg(_N_CLS // _NC)


def _layer_tc_body(p_ref, c_ref, w0_ref, b0_ref, w1_ref, b1_ref, z_ref):
    ssum = jnp.concatenate([p_ref[0], p_ref[1]], axis=-1)
    cntv = c_ref[0, :, 0:1] + c_ref[1, :, 0:1]
    mean = ssum / jnp.maximum(cntv, 1.0)
    mask = (cntv > 0.0).astype(jnp.float32)
    h = jnp.dot(mean, w0_ref[...], preferred_element_type=jnp.float32)
    h = jnp.maximum(h + b0_ref[...] * mask, 0.0)
    z2 = (jnp.dot(h, w1_ref[...], preferred_element_type=jnp.float32)
          + b1_ref[...])
    half = _N_CLS // _NC
    z_ref[0] = z2[:, :half]
    z_ref[1] = z2[:, half:]


def _mean_tc_body(p_ref, c_ref, o_ref):
    ssum = jnp.concatenate([p_ref[0], p_ref[1]], axis=-1)
    cntv = c_ref[0, :, 0:1] + c_ref[1, :, 0:1]
    o_ref[...] = ssum / jnp.maximum(cntv, 1.0)


_BM = 1000


def _layer_tc(p, c, w0, b0, w1, b1):
    return pl.pallas_call(
        _layer_tc_body,
        grid=(_N // _BM,),
        in_specs=[
            pl.BlockSpec((_NC, _BM, _D_IN // _NC), lambda i: (0, i, 0)),
            pl.BlockSpec((_NC, _BM, 16), lambda i: (0, i, 0)),
            pl.BlockSpec((_D_IN, _D_H), lambda i: (0, 0)),
            pl.BlockSpec((1, _D_H), lambda i: (0, 0)),
            pl.BlockSpec((_D_H, _N_CLS), lambda i: (0, 0)),
            pl.BlockSpec((1, _N_CLS), lambda i: (0, 0)),
        ],
        out_specs=pl.BlockSpec((_NC, _BM, _N_CLS // _NC), lambda i: (0, i, 0)),
        out_shape=jax.ShapeDtypeStruct((_NC, _N, _N_CLS // _NC), jnp.float32),
        name="sage_dense",
    )(p, c, w0, b0, w1, b1)


def _mean_tc(p, c):
    return pl.pallas_call(
        _mean_tc_body,
        grid=(_N // _BM,),
        in_specs=[
            pl.BlockSpec((_NC, _BM, _N_CLS // _NC), lambda i: (0, i, 0)),
            pl.BlockSpec((_NC, _BM, 16), lambda i: (0, i, 0)),
        ],
        out_specs=pl.BlockSpec((_BM, _N_CLS), lambda i: (i, 0)),
        out_shape=jax.ShapeDtypeStruct((_N, _N_CLS), jnp.float32),
        name="sage_mean",
    )(p, c)


def kernel(x, edge_index0, edge_index1, W0, b0, W1, b1):
    z64 = jnp.zeros((_N_PAD, _D_IN // _NC), jnp.float32)
    z32 = jnp.zeros((_N_PAD, _N_CLS // _NC), jnp.float32)
    z16 = jnp.zeros((_N_PAD, 16), jnp.float32)
    ones16 = jnp.ones((_CH, 16), jnp.float32)

    d2 = _D_IN // _NC
    x3 = jnp.stack([x[:, :d2], x[:, d2:]])
    e0r = edge_index0.reshape(2, _E // _CH, _CH)
    e1r = edge_index1.reshape(2, _E // _CH, _CH)
    p0, c0 = _agg64(x3, e0r, z64, z16, ones16)
    z3 = _layer_tc(p0, c0, W0, b0.reshape(1, -1), W1, b1.reshape(1, -1))
    p1, c1 = _agg32(z3, e1r, z32, z16, ones16)
    return _mean_tc(p1, c1)


# speedup vs baseline: 1.1338x; 1.0523x over previous
import functools

import jax
import jax.numpy as jnp
from jax import lax
from jax.experimental import pallas as pl
from jax.experimental.pallas import tpu as pltpu
from jax.experimental.pallas import tpu_sc as plsc

_N = 10000
_D_IN = 128
_D_H = 256
_N_CLS = 64
_E = 320000

_NC = 2
_NS = 16
_CH = 128
_SUB = 4
_CPT = 156
_BLOCKS = _CPT // _SUB
_HALFB = _BLOCKS // 2
_XTRA = _E // _CH - _NS * _CPT
_N_PAD = 10240
_OPT = 624


def _agg_body(table, edges, zrows, z16, ones16, out_sum, out_cnt,
              src_v, dst_v, rows_v, ones_v, acc, cnt, sem0, sem1, sem2,
              *, d2):
    c = lax.axis_index("c")
    s = lax.axis_index("s")
    wid = c * _NS + s
    sems = (sem0, sem1)

    rpt = _N_PAD // _NS
    r0 = s * rpt
    pltpu.sync_copy(zrows.at[pl.ds(r0, rpt)], acc.at[pl.ds(r0, rpt)])
    pltpu.sync_copy(z16.at[pl.ds(r0, rpt)], cnt.at[pl.ds(r0, rpt)])
    pltpu.sync_copy(ones16, ones_v)
    plsc.subcore_barrier()

    chunk0 = s * _CPT
    tbl = table.at[c]

    def _fetch(k, blk, nch, chunk_base):
        pltpu.sync_copy(edges.at[0, pl.ds(chunk_base, nch)],
                        src_v.at[k, pl.ds(0, nch)])
        pltpu.sync_copy(edges.at[1, pl.ds(chunk_base, nch)],
                        dst_v.at[k, pl.ds(0, nch)])
        for j in range(nch):
            pltpu.async_copy(tbl.at[src_v.at[k, j]], rows_v.at[k, j],
                             sems[k])

    def _drain_scatter(k, nch, docnt):
        for j in range(nch):
            pltpu.make_async_copy(tbl.at[src_v.at[k, j]],
                                  rows_v.at[k, j], sems[k]).wait()
        for j in range(nch):
            pltpu.async_copy(rows_v.at[k, j], acc.at[dst_v.at[k, j]], sem2,
                             add=True)

        @pl.when(docnt)
        def _():
            for j in range(nch):
                pltpu.async_copy(ones_v, cnt.at[dst_v.at[k, j]], sem2,
                                 add=True)
            for j in range(nch):
                pltpu.make_async_copy(ones_v, cnt.at[dst_v.at[k, j]],
                                      sem2).wait()
        for j in range(nch):
            pltpu.make_async_copy(rows_v.at[k, j], acc.at[dst_v.at[k, j]],
                                  sem2).wait()

    for k in range(2):
        _fetch(k, k, _SUB, chunk0 + k * _SUB)

    def body(io, carry):
        for k in range(2):
            i = io * 2 + k
            docnt = jnp.where(c == 0, i < _HALFB, i >= _HALFB)
            _drain_scatter(k, _SUB, docnt)

            @pl.when(i + 2 < _BLOCKS)
            def _():
                _fetch(k, i + 2, _SUB, chunk0 + (i + 2) * _SUB)
        return carry

    lax.fori_loop(0, (_BLOCKS - 1) // 2, body, 0)
    _drain_scatter(0, _SUB, c == 1)

    @pl.when(s == 0)
    def _():
        _fetch(0, 0, _XTRA, _NS * _CPT)
        _drain_scatter(0, _XTRA, c == 1)

    plsc.subcore_barrier()

    o0 = s * _OPT
    rem0 = (_NS - 1) * _OPT
    rem = _N - rem0

    @pl.when(s < _NS - 1)
    def _():
        pltpu.sync_copy(acc.at[pl.ds(o0, _OPT)], out_sum.at[c, pl.ds(o0, _OPT)])
        pltpu.sync_copy(cnt.at[pl.ds(o0, _OPT)], out_cnt.at[c, pl.ds(o0, _OPT)])

    @pl.when(s == _NS - 1)
    def _():
        pltpu.sync_copy(acc.at[pl.ds(rem0, rem)], out_sum.at[c, pl.ds(rem0, rem)])
        pltpu.sync_copy(cnt.at[pl.ds(rem0, rem)], out_cnt.at[c, pl.ds(rem0, rem)])


def _make_agg(d2):
    mesh = plsc.VectorSubcoreMesh(core_axis_name="c", subcore_axis_name="s",
                                  num_cores=_NC, num_subcores=_NS)
    return functools.partial(
        pl.kernel,
        mesh=mesh,
        out_type=[jax.ShapeDtypeStruct((_NC, _N, d2), jnp.float32),
                  jax.ShapeDtypeStruct((_NC, _N, 16), jnp.float32)],
        scratch_types=[
            pltpu.VMEM((2, _SUB, _CH), jnp.int32),
            pltpu.VMEM((2, _SUB, _CH), jnp.int32),
            pltpu.VMEM((2, _SUB, _CH, d2), jnp.float32),
            pltpu.VMEM((_CH, 16), jnp.float32),
            pltpu.VMEM_SHARED((_N_PAD, d2), jnp.float32),
            pltpu.VMEM_SHARED((_N_PAD, 16), jnp.float32),
            pltpu.SemaphoreType.DMA,
            pltpu.SemaphoreType.DMA,
            pltpu.SemaphoreType.DMA,
        ],
        compiler_params=pltpu.CompilerParams(use_tc_tiling_on_sc=False),
        name=f"sage_segsum_d{d2}",
    )(functools.partial(_agg_body, d2=d2))


_agg64 = _make_agg(_D_IN // _NC)
_agg32 = _make_agg(_N_CLS // _NC)


def _layer_tc_body(p_ref, c_ref, w0_ref, b0_ref, w1_ref, b1_ref, z_ref):
    ssum = jnp.concatenate([p_ref[0], p_ref[1]], axis=-1)
    cntv = c_ref[0, :, 0:1] + c_ref[1, :, 0:1]
    mean = ssum / jnp.maximum(cntv, 1.0)
    mask = (cntv > 0.0).astype(jnp.float32)
    h = jnp.dot(mean, w0_ref[...], preferred_element_type=jnp.float32)
    h = jnp.maximum(h + b0_ref[...] * mask, 0.0)
    z2 = (jnp.dot(h, w1_ref[...], preferred_element_type=jnp.float32)
          + b1_ref[...])
    half = _N_CLS // _NC
    z_ref[0] = z2[:, :half]
    z_ref[1] = z2[:, half:]


def _mean_tc_body(p_ref, c_ref, o_ref):
    ssum = jnp.concatenate([p_ref[0], p_ref[1]], axis=-1)
    cntv = c_ref[0, :, 0:1] + c_ref[1, :, 0:1]
    o_ref[...] = ssum / jnp.maximum(cntv, 1.0)


_BM = 1000


def _layer_tc(p, c, w0, b0, w1, b1):
    return pl.pallas_call(
        _layer_tc_body,
        grid=(_N // _BM,),
        in_specs=[
            pl.BlockSpec((_NC, _BM, _D_IN // _NC), lambda i: (0, i, 0)),
            pl.BlockSpec((_NC, _BM, 16), lambda i: (0, i, 0)),
            pl.BlockSpec((_D_IN, _D_H), lambda i: (0, 0)),
            pl.BlockSpec((1, _D_H), lambda i: (0, 0)),
            pl.BlockSpec((_D_H, _N_CLS), lambda i: (0, 0)),
            pl.BlockSpec((1, _N_CLS), lambda i: (0, 0)),
        ],
        out_specs=pl.BlockSpec((_NC, _BM, _N_CLS // _NC), lambda i: (0, i, 0)),
        out_shape=jax.ShapeDtypeStruct((_NC, _N, _N_CLS // _NC), jnp.float32),
        name="sage_dense",
    )(p, c, w0, b0, w1, b1)


def _mean_tc(p, c):
    return pl.pallas_call(
        _mean_tc_body,
        grid=(_N // _BM,),
        in_specs=[
            pl.BlockSpec((_NC, _BM, _N_CLS // _NC), lambda i: (0, i, 0)),
            pl.BlockSpec((_NC, _BM, 16), lambda i: (0, i, 0)),
        ],
        out_specs=pl.BlockSpec((_BM, _N_CLS), lambda i: (i, 0)),
        out_shape=jax.ShapeDtypeStruct((_N, _N_CLS), jnp.float32),
        name="sage_mean",
    )(p, c)


def kernel(x, edge_index0, edge_index1, W0, b0, W1, b1):
    z64 = jnp.zeros((_N_PAD, _D_IN // _NC), jnp.float32)
    z32 = jnp.zeros((_N_PAD, _N_CLS // _NC), jnp.float32)
    z16 = jnp.zeros((_N_PAD, 16), jnp.float32)
    ones16 = jnp.ones((_CH, 16), jnp.float32)

    d2 = _D_IN // _NC
    x3 = jnp.stack([x[:, :d2], x[:, d2:]])
    e0r = edge_index0.reshape(2, _E // _CH, _CH)
    e1r = edge_index1.reshape(2, _E // _CH, _CH)
    p0, c0 = _agg64(x3, e0r, z64, z16, ones16)
    z3 = _layer_tc(p0, c0, W0, b0.reshape(1, -1), W1, b1.reshape(1, -1))
    p1, c1 = _agg32(z3, e1r, z32, z16, ones16)
    return _mean_tc(p1, c1)


# speedup vs baseline: 1.1339x; 1.0002x over previous
import functools

import jax
import jax.numpy as jnp
from jax import lax
from jax.experimental import pallas as pl
from jax.experimental.pallas import tpu as pltpu
from jax.experimental.pallas import tpu_sc as plsc

_N = 10000
_D_IN = 128
_D_H = 256
_N_CLS = 64
_E = 320000

_NC = 2
_NS = 16
_CH = 128
_SUB = 4
_NBUF = 2
_CPT = 156
_BLOCKS = _CPT // _SUB
_HALFB = _BLOCKS // 2
_XTRA = _E // _CH - _NS * _CPT
_N_PAD = 10240
_OPT = 624


def _agg_body(table, edges, zrows, z16, ones16, out_sum, out_cnt,
              src_v, dst_v, rows_v, ones_v, acc, cnt, sem0, sem1, sem2,
              *, d2):
    c = lax.axis_index("c")
    s = lax.axis_index("s")
    sems = (sem0, sem1)

    rpt = _N_PAD // _NS
    r0 = s * rpt
    pltpu.sync_copy(zrows.at[pl.ds(r0, rpt)], acc.at[pl.ds(r0, rpt)])
    pltpu.sync_copy(z16.at[pl.ds(r0, rpt)], cnt.at[pl.ds(r0, rpt)])
    pltpu.sync_copy(ones16, ones_v)
    plsc.subcore_barrier()

    chunk0 = s * _CPT
    tbl = table.at[c]

    def _fetch(k, blk, nch, chunk_base):
        pltpu.sync_copy(edges.at[0, pl.ds(chunk_base, nch)],
                        src_v.at[k, pl.ds(0, nch)])
        pltpu.sync_copy(edges.at[1, pl.ds(chunk_base, nch)],
                        dst_v.at[k, pl.ds(0, nch)])
        for j in range(nch):
            pltpu.async_copy(tbl.at[src_v.at[k, j]], rows_v.at[k, j],
                             sems[k])

    def _drain_scatter(k, nch, docnt):
        for j in range(nch):
            pltpu.make_async_copy(tbl.at[src_v.at[k, j]],
                                  rows_v.at[k, j], sems[k]).wait()
        for j in range(nch):
            pltpu.async_copy(rows_v.at[k, j], acc.at[dst_v.at[k, j]], sem2,
                             add=True)

        @pl.when(docnt)
        def _():
            for j in range(nch):
                pltpu.async_copy(ones_v, cnt.at[dst_v.at[k, j]], sem2,
                                 add=True)
            for j in range(nch):
                pltpu.make_async_copy(ones_v, cnt.at[dst_v.at[k, j]],
                                      sem2).wait()
        for j in range(nch):
            pltpu.make_async_copy(rows_v.at[k, j], acc.at[dst_v.at[k, j]],
                                  sem2).wait()

    for k in range(_NBUF):
        _fetch(k, k, _SUB, chunk0 + k * _SUB)

    def body(io, carry):
        for k in range(_NBUF):
            i = io * _NBUF + k
            docnt = jnp.where(c == 0, i < _HALFB, i >= _HALFB)
            _drain_scatter(k, _SUB, docnt)

            @pl.when(i + _NBUF < _BLOCKS)
            def _():
                _fetch(k, i + _NBUF, _SUB, chunk0 + (i + _NBUF) * _SUB)
        return carry

    lax.fori_loop(0, (_BLOCKS - 1) // _NBUF, body, 0)
    _drain_scatter(0, _SUB, c == 1)

    @pl.when(s == 0)
    def _():
        _fetch(0, 0, _XTRA, _NS * _CPT)
        _drain_scatter(0, _XTRA, c == 1)

    plsc.subcore_barrier()

    o0 = s * _OPT
    rem0 = (_NS - 1) * _OPT
    rem = _N - rem0

    @pl.when(s < _NS - 1)
    def _():
        pltpu.sync_copy(acc.at[pl.ds(o0, _OPT)], out_sum.at[c, pl.ds(o0, _OPT)])
        pltpu.sync_copy(cnt.at[pl.ds(o0, _OPT)], out_cnt.at[c, pl.ds(o0, _OPT)])

    @pl.when(s == _NS - 1)
    def _():
        pltpu.sync_copy(acc.at[pl.ds(rem0, rem)], out_sum.at[c, pl.ds(rem0, rem)])
        pltpu.sync_copy(cnt.at[pl.ds(rem0, rem)], out_cnt.at[c, pl.ds(rem0, rem)])


def _make_agg(d2):
    mesh = plsc.VectorSubcoreMesh(core_axis_name="c", subcore_axis_name="s",
                                  num_cores=_NC, num_subcores=_NS)
    return functools.partial(
        pl.kernel,
        mesh=mesh,
        out_type=[jax.ShapeDtypeStruct((_NC, _N, d2), jnp.float32),
                  jax.ShapeDtypeStruct((_NC, _N, 16), jnp.float32)],
        scratch_types=[
            pltpu.VMEM((_NBUF, _SUB, _CH), jnp.int32),
            pltpu.VMEM((_NBUF, _SUB, _CH), jnp.int32),
            pltpu.VMEM((_NBUF, _SUB, _CH, d2), jnp.float32),
            pltpu.VMEM((_CH, 16), jnp.float32),
            pltpu.VMEM_SHARED((_N_PAD, d2), jnp.float32),
            pltpu.VMEM_SHARED((_N_PAD, 16), jnp.float32),
            pltpu.SemaphoreType.DMA,
            pltpu.SemaphoreType.DMA,
            pltpu.SemaphoreType.DMA,
        ],
        compiler_params=pltpu.CompilerParams(use_tc_tiling_on_sc=False),
        name=f"sage_segsum_d{d2}",
    )(functools.partial(_agg_body, d2=d2))


_agg64 = _make_agg(_D_IN // _NC)
_agg32 = _make_agg(_N_CLS // _NC)


def _layer_tc_body(p_ref, c_ref, w0_ref, b0_ref, w1_ref, b1_ref, z_ref):
    ssum = jnp.concatenate([p_ref[0], p_ref[1]], axis=-1)
    cntv = c_ref[0, :, 0:1] + c_ref[1, :, 0:1]
    mean = ssum / jnp.maximum(cntv, 1.0)
    mask = (cntv > 0.0).astype(jnp.float32)
    h = jnp.dot(mean, w0_ref[...], preferred_element_type=jnp.float32)
    h = jnp.maximum(h + b0_ref[...] * mask, 0.0)
    z2 = (jnp.dot(h, w1_ref[...], preferred_element_type=jnp.float32)
          + b1_ref[...])
    half = _N_CLS // _NC
    z_ref[0] = z2[:, :half]
    z_ref[1] = z2[:, half:]


def _mean_tc_body(p_ref, c_ref, o_ref):
    ssum = jnp.concatenate([p_ref[0], p_ref[1]], axis=-1)
    cntv = c_ref[0, :, 0:1] + c_ref[1, :, 0:1]
    o_ref[...] = ssum / jnp.maximum(cntv, 1.0)


_BM = 1000


def _layer_tc(p, c, w0, b0, w1, b1):
    return pl.pallas_call(
        _layer_tc_body,
        grid=(_N // _BM,),
        in_specs=[
            pl.BlockSpec((_NC, _BM, _D_IN // _NC), lambda i: (0, i, 0)),
            pl.BlockSpec((_NC, _BM, 16), lambda i: (0, i, 0)),
            pl.BlockSpec((_D_IN, _D_H), lambda i: (0, 0)),
            pl.BlockSpec((1, _D_H), lambda i: (0, 0)),
            pl.BlockSpec((_D_H, _N_CLS), lambda i: (0, 0)),
            pl.BlockSpec((1, _N_CLS), lambda i: (0, 0)),
        ],
        out_specs=pl.BlockSpec((_NC, _BM, _N_CLS // _NC), lambda i: (0, i, 0)),
        out_shape=jax.ShapeDtypeStruct((_NC, _N, _N_CLS // _NC), jnp.float32),
        name="sage_dense",
    )(p, c, w0, b0, w1, b1)


def _mean_tc(p, c):
    return pl.pallas_call(
        _mean_tc_body,
        grid=(_N // _BM,),
        in_specs=[
            pl.BlockSpec((_NC, _BM, _N_CLS // _NC), lambda i: (0, i, 0)),
            pl.BlockSpec((_NC, _BM, 16), lambda i: (0, i, 0)),
        ],
        out_specs=pl.BlockSpec((_BM, _N_CLS), lambda i: (i, 0)),
        out_shape=jax.ShapeDtypeStruct((_N, _N_CLS), jnp.float32),
        name="sage_mean",
    )(p, c)


def kernel(x, edge_index0, edge_index1, W0, b0, W1, b1):
    z64 = jnp.zeros((_N_PAD, _D_IN // _NC), jnp.float32)
    z32 = jnp.zeros((_N_PAD, _N_CLS // _NC), jnp.float32)
    z16 = jnp.zeros((_N_PAD, 16), jnp.float32)
    ones16 = jnp.ones((_CH, 16), jnp.float32)

    e0r = edge_index0.reshape(2, _E // _CH, _CH)
    e1r = edge_index1.reshape(2, _E // _CH, _CH)
    d2 = _D_IN // _NC
    x3 = jnp.stack([x[:, :d2], x[:, d2:]])
    p0, c0 = _agg64(x3, e0r, z64, z16, ones16)
    z3 = _layer_tc(p0, c0, W0, b0.reshape(1, -1), W1, b1.reshape(1, -1))
    p1, c1 = _agg32(z3, e1r, z32, z16, ones16)
    return _mean_tc(p1, c1)


# speedup vs baseline: 1.1469x; 1.0114x over previous
import functools

import jax
import jax.numpy as jnp
from jax import lax
from jax.experimental import pallas as pl
from jax.experimental.pallas import tpu as pltpu
from jax.experimental.pallas import tpu_sc as plsc

_N = 10000
_D_IN = 128
_D_H = 256
_N_CLS = 64
_E = 320000

_NC = 2
_NS = 16
_CH = 128
_SUB = 4
_NBUF = 2
_CPT = 156
_BLOCKS = _CPT // _SUB
_HALFB = _BLOCKS // 2
_XTRA = _E // _CH - _NS * _CPT
_N_PAD = 10240
_OPT = 624


def _agg_body(table, edges, zrows, z16, ones16, out_sum, out_cnt,
              src_v, dst_v, rows_v, ones_v, acc, cnt, sem0, sem1, sem2,
              *, d2):
    c = lax.axis_index("c")
    s = lax.axis_index("s")
    sems = (sem0, sem1)

    rpt = _N_PAD // _NS
    r0 = s * rpt
    pltpu.sync_copy(zrows.at[pl.ds(r0, rpt)], acc.at[pl.ds(r0, rpt)])
    pltpu.sync_copy(z16.at[pl.ds(r0, rpt)], cnt.at[pl.ds(r0, rpt)])
    pltpu.sync_copy(ones16, ones_v)
    plsc.subcore_barrier()

    chunk0 = s * _CPT
    tbl = table.at[c]

    def _fetch(k, blk, nch, chunk_base):
        pltpu.sync_copy(edges.at[0, pl.ds(chunk_base, nch)],
                        src_v.at[k, pl.ds(0, nch)])
        pltpu.sync_copy(edges.at[1, pl.ds(chunk_base, nch)],
                        dst_v.at[k, pl.ds(0, nch)])
        for j in range(nch):
            pltpu.async_copy(tbl.at[src_v.at[k, j]], rows_v.at[k, j],
                             sems[k])

    def _drain_scatter(k, nch, docnt):
        for j in range(nch):
            pltpu.make_async_copy(tbl.at[src_v.at[k, j]],
                                  rows_v.at[k, j], sems[k]).wait()
        for j in range(nch):
            pltpu.async_copy(rows_v.at[k, j], acc.at[dst_v.at[k, j]], sem2,
                             add=True)

        @pl.when(docnt)
        def _():
            for j in range(nch):
                pltpu.async_copy(ones_v, cnt.at[dst_v.at[k, j]], sem2,
                                 add=True)
            for j in range(nch):
                pltpu.make_async_copy(ones_v, cnt.at[dst_v.at[k, j]],
                                      sem2).wait()
        for j in range(nch):
            pltpu.make_async_copy(rows_v.at[k, j], acc.at[dst_v.at[k, j]],
                                  sem2).wait()

    for k in range(_NBUF):
        _fetch(k, k, _SUB, chunk0 + k * _SUB)

    def body(io, carry):
        for k in range(_NBUF):
            i = io * _NBUF + k
            docnt = jnp.where(c == 0, i < _HALFB, i >= _HALFB)
            _drain_scatter(k, _SUB, docnt)

            @pl.when(i + _NBUF < _BLOCKS)
            def _():
                _fetch(k, i + _NBUF, _SUB, chunk0 + (i + _NBUF) * _SUB)
        return carry

    lax.fori_loop(0, (_BLOCKS - 1) // _NBUF, body, 0)
    _drain_scatter(0, _SUB, c == 1)

    @pl.when(s == 0)
    def _():
        _fetch(0, 0, _XTRA, _NS * _CPT)
        _drain_scatter(0, _XTRA, c == 1)

    plsc.subcore_barrier()

    o0 = s * _OPT
    rem0 = (_NS - 1) * _OPT
    rem = _N - rem0

    @pl.when(s < _NS - 1)
    def _():
        pltpu.sync_copy(acc.at[pl.ds(o0, _OPT)], out_sum.at[c, pl.ds(o0, _OPT)])
        pltpu.sync_copy(cnt.at[pl.ds(o0, _OPT)], out_cnt.at[c, pl.ds(o0, _OPT)])

    @pl.when(s == _NS - 1)
    def _():
        pltpu.sync_copy(acc.at[pl.ds(rem0, rem)], out_sum.at[c, pl.ds(rem0, rem)])
        pltpu.sync_copy(cnt.at[pl.ds(rem0, rem)], out_cnt.at[c, pl.ds(rem0, rem)])


def _make_agg(d2):
    mesh = plsc.VectorSubcoreMesh(core_axis_name="c", subcore_axis_name="s",
                                  num_cores=_NC, num_subcores=_NS)
    return functools.partial(
        pl.kernel,
        mesh=mesh,
        out_type=[jax.ShapeDtypeStruct((_NC, _N, d2), jnp.float32),
                  jax.ShapeDtypeStruct((_NC, _N, 16), jnp.float32)],
        scratch_types=[
            pltpu.VMEM((_NBUF, _SUB, _CH), jnp.int32),
            pltpu.VMEM((_NBUF, _SUB, _CH), jnp.int32),
            pltpu.VMEM((_NBUF, _SUB, _CH, d2), jnp.float32),
            pltpu.VMEM((_CH, 16), jnp.float32),
            pltpu.VMEM_SHARED((_N_PAD, d2), jnp.float32),
            pltpu.VMEM_SHARED((_N_PAD, 16), jnp.float32),
            pltpu.SemaphoreType.DMA,
            pltpu.SemaphoreType.DMA,
            pltpu.SemaphoreType.DMA,
        ],
        compiler_params=pltpu.CompilerParams(use_tc_tiling_on_sc=False),
        name=f"sage_segsum_d{d2}",
    )(functools.partial(_agg_body, d2=d2))


_agg64 = _make_agg(_D_IN // _NC)
_agg32 = _make_agg(_N_CLS // _NC)


def _layer_tc_body(p_ref, c_ref, w0_ref, b0_ref, w1_ref, b1_ref, z_ref):
    ssum = jnp.concatenate([p_ref[0], p_ref[1]], axis=-1)
    cntv = c_ref[0, :, 0:1] + c_ref[1, :, 0:1]
    mean = ssum / jnp.maximum(cntv, 1.0)
    mask = (cntv > 0.0).astype(jnp.float32)
    h = jnp.dot(mean, w0_ref[...], preferred_element_type=jnp.float32)
    h = jnp.maximum(h + b0_ref[...] * mask, 0.0)
    z2 = (jnp.dot(h, w1_ref[...], preferred_element_type=jnp.float32)
          + b1_ref[...])
    half = _N_CLS // _NC
    z_ref[0] = z2[:, :half]
    z_ref[1] = z2[:, half:]


def _mean_tc_body(p_ref, c_ref, o_ref):
    ssum = jnp.concatenate([p_ref[0], p_ref[1]], axis=-1)
    cntv = c_ref[0, :, 0:1] + c_ref[1, :, 0:1]
    o_ref[...] = ssum / jnp.maximum(cntv, 1.0)


_BM = 2000


def _layer_tc(p, c, w0, b0, w1, b1):
    return pl.pallas_call(
        _layer_tc_body,
        grid=(_N // _BM,),
        in_specs=[
            pl.BlockSpec((_NC, _BM, _D_IN // _NC), lambda i: (0, i, 0)),
            pl.BlockSpec((_NC, _BM, 16), lambda i: (0, i, 0)),
            pl.BlockSpec((_D_IN, _D_H), lambda i: (0, 0)),
            pl.BlockSpec((1, _D_H), lambda i: (0, 0)),
            pl.BlockSpec((_D_H, _N_CLS), lambda i: (0, 0)),
            pl.BlockSpec((1, _N_CLS), lambda i: (0, 0)),
        ],
        out_specs=pl.BlockSpec((_NC, _BM, _N_CLS // _NC), lambda i: (0, i, 0)),
        out_shape=jax.ShapeDtypeStruct((_NC, _N, _N_CLS // _NC), jnp.float32),
        name="sage_dense",
    )(p, c, w0, b0, w1, b1)


def _mean_tc(p, c):
    return pl.pallas_call(
        _mean_tc_body,
        grid=(_N // _BM,),
        in_specs=[
            pl.BlockSpec((_NC, _BM, _N_CLS // _NC), lambda i: (0, i, 0)),
            pl.BlockSpec((_NC, _BM, 16), lambda i: (0, i, 0)),
        ],
        out_specs=pl.BlockSpec((_BM, _N_CLS), lambda i: (i, 0)),
        out_shape=jax.ShapeDtypeStruct((_N, _N_CLS), jnp.float32),
        name="sage_mean",
    )(p, c)


def kernel(x, edge_index0, edge_index1, W0, b0, W1, b1):
    z64 = jnp.zeros((_N_PAD, _D_IN // _NC), jnp.float32)
    z32 = jnp.zeros((_N_PAD, _N_CLS // _NC), jnp.float32)
    z16 = jnp.zeros((_N_PAD, 16), jnp.float32)
    ones16 = jnp.ones((_CH, 16), jnp.float32)

    e0r = edge_index0.reshape(2, _E // _CH, _CH)
    e1r = edge_index1.reshape(2, _E // _CH, _CH)
    d2 = _D_IN // _NC
    x3 = jnp.stack([x[:, :d2], x[:, d2:]])
    p0, c0 = _agg64(x3, e0r, z64, z16, ones16)
    z3 = _layer_tc(p0, c0, W0, b0.reshape(1, -1), W1, b1.reshape(1, -1))
    p1, c1 = _agg32(z3, e1r, z32, z16, ones16)
    return _mean_tc(p1, c1)
